# baseline (speedup 1.0000x reference)
import functools

import jax
import jax.numpy as jnp
from jax import lax
from jax.experimental import pallas as pl
from jax.experimental.pallas import tpu as pltpu
from jax.experimental.pallas import tpu_sc as plsc

D_MODEL = 128
CHUNK = 128
NBUF = 5


@functools.cache
def _build(num_rows: int):
    info = plsc.get_sparse_core_info()
    nc, ns = info.num_cores, info.num_subcores
    nw = nc * ns
    assert num_rows % (nw * CHUNK * NBUF) == 0
    chunks_per_w = num_rows // (nw * CHUNK)
    rows_per_w = chunks_per_w * CHUNK
    ngroups = chunks_per_w // NBUF

    mesh = plsc.VectorSubcoreMesh(core_axis_name="c", subcore_axis_name="s")

    @functools.partial(
        pl.kernel,
        out_type=jax.ShapeDtypeStruct((num_rows, D_MODEL), jnp.float32),
        mesh=mesh,
        scratch_types=[
            pltpu.VMEM((chunks_per_w, CHUNK), jnp.int32),
            [pltpu.VMEM((CHUNK, D_MODEL), jnp.float32) for _ in range(NBUF)],
            [pltpu.SemaphoreType.DMA for _ in range(NBUF)],
            [pltpu.SemaphoreType.DMA for _ in range(NBUF)],
        ],
    )
    def gather_kernel(pe_hbm, idx_hbm, out_hbm, idx_v, rows, gsem, ssem):
        wid = lax.axis_index("s") * nc + lax.axis_index("c")
        base = wid * rows_per_w
        pltpu.sync_copy(idx_hbm.at[wid], idx_v)

        def gather(c, b):
            pltpu.async_copy(pe_hbm.at[idx_v.at[c]], rows[b], gsem[b])

        def wait_gather(b):
            pltpu.make_async_copy(pe_hbm.at[idx_v.at[0]], rows[b], gsem[b]).wait()

        def scatter(c, b):
            pltpu.async_copy(
                rows[b], out_hbm.at[pl.ds(base + c * CHUNK, CHUNK)], ssem[b]
            )

        def wait_scatter(b):
            pltpu.make_async_copy(
                rows[b], out_hbm.at[pl.ds(base, CHUNK)], ssem[b]
            ).wait()

        for b in range(NBUF):
            gather(b, b)

        def body(i, _):
            c0 = i * NBUF
            for b in range(NBUF):
                wait_gather(b)
                gather(c0 + NBUF + b, b)
            return _

        lax.fori_loop(0, ngroups - 1, body, 0)
        for b in range(NBUF):
            wait_gather(b)
        scatter(0, 0)
        wait_scatter(0)

    def run(pe, idx_flat):
        idx3 = idx_flat.reshape(nw, chunks_per_w, CHUNK)
        return gather_kernel(pe, idx3)

    return run


@jax.jit
def kernel(time, pe):
    b, l = time.shape
    idx_flat = time.astype(jnp.int32).reshape(b * l)
    out = _build(b * l)(pe, idx_flat)
    return out.reshape(b, l, D_MODEL)


# speedup vs baseline: 2.0718x; 2.0718x over previous
import functools

import jax
import jax.numpy as jnp
from jax import lax
from jax.experimental import pallas as pl
from jax.experimental.pallas import tpu as pltpu
from jax.experimental.pallas import tpu_sc as plsc

D_MODEL = 128
TABLE_ROWS = 367
CHUNK = 128
NBUF = 5


@functools.cache
def _build(num_rows: int):
    info = plsc.get_sparse_core_info()
    nc, ns = info.num_cores, info.num_subcores
    nw = nc * ns
    assert num_rows % (nw * CHUNK * NBUF) == 0
    chunks_per_w = num_rows // (nw * CHUNK)
    rows_per_w = chunks_per_w * CHUNK
    ngroups = chunks_per_w // NBUF

    mesh = plsc.VectorSubcoreMesh(core_axis_name="c", subcore_axis_name="s")

    @functools.partial(
        pl.kernel,
        out_type=jax.ShapeDtypeStruct((num_rows, D_MODEL), jnp.float32),
        mesh=mesh,
        scratch_types=[
            pltpu.VMEM_SHARED((TABLE_ROWS, D_MODEL), jnp.float32),
            pltpu.VMEM((chunks_per_w, CHUNK), jnp.int32),
            [pltpu.VMEM((CHUNK, D_MODEL), jnp.float32) for _ in range(NBUF)],
            [pltpu.SemaphoreType.DMA for _ in range(NBUF)],
            [pltpu.SemaphoreType.DMA for _ in range(NBUF)],
        ],
    )
    def gather_kernel(pe_hbm, idx_hbm, out_hbm, tbl_sp, idx_v, rows, gsem, ssem):
        wid = lax.axis_index("s") * nc + lax.axis_index("c")
        base = wid * rows_per_w

        @pl.when(lax.axis_index("s") == 0)
        def _stage():
            pltpu.sync_copy(pe_hbm, tbl_sp)

        pltpu.sync_copy(idx_hbm.at[wid], idx_v)
        plsc.subcore_barrier()

        def gather(c, b):
            pltpu.async_copy(tbl_sp.at[idx_v.at[c]], rows[b], gsem[b])

        def wait_gather(b):
            pltpu.make_async_copy(tbl_sp.at[idx_v.at[0]], rows[b], gsem[b]).wait()

        def scatter(c, b):
            pltpu.async_copy(
                rows[b], out_hbm.at[pl.ds(base + c * CHUNK, CHUNK)], ssem[b]
            )

        def wait_scatter(b):
            pltpu.make_async_copy(
                rows[b], out_hbm.at[pl.ds(base, CHUNK)], ssem[b]
            ).wait()

        for b in range(NBUF):
            gather(b, b)

        def body(i, _):
            c0 = i * NBUF
            for b in range(NBUF):
                wait_gather(b)
                scatter(c0 + b, b)
            for b in range(NBUF):
                wait_scatter(b)
                gather(c0 + NBUF + b, b)
            return _

        lax.fori_loop(0, ngroups - 1, body, 0)

        for b in range(NBUF):
            wait_gather(b)
            scatter((ngroups - 1) * NBUF + b, b)
        for b in range(NBUF):
            wait_scatter(b)

    def run(pe, idx_flat):
        idx3 = idx_flat.reshape(nw, chunks_per_w, CHUNK)
        return gather_kernel(pe, idx3)

    return run


@jax.jit
def kernel(time, pe):
    b, l = time.shape
    idx_flat = time.astype(jnp.int32).reshape(b * l)
    out = _build(b * l)(pe, idx_flat)
    return out.reshape(b, l, D_MODEL)


# speedup vs baseline: 2.1125x; 1.0197x over previous
import functools

import jax
import jax.numpy as jnp
from jax import lax
from jax.experimental import pallas as pl
from jax.experimental.pallas import tpu as pltpu
from jax.experimental.pallas import tpu_sc as plsc

D_MODEL = 128
TABLE_ROWS = 367
CHUNK = 80
NBUF = 8


@functools.cache
def _build(num_rows: int):
    info = plsc.get_sparse_core_info()
    nc, ns = info.num_cores, info.num_subcores
    nw = nc * ns
    assert num_rows % (nw * CHUNK * NBUF) == 0
    chunks_per_w = num_rows // (nw * CHUNK)
    rows_per_w = chunks_per_w * CHUNK
    ngroups = chunks_per_w // NBUF

    mesh = plsc.VectorSubcoreMesh(core_axis_name="c", subcore_axis_name="s")

    @functools.partial(
        pl.kernel,
        out_type=jax.ShapeDtypeStruct((num_rows, D_MODEL), jnp.float32),
        mesh=mesh,
        scratch_types=[
            pltpu.VMEM_SHARED((TABLE_ROWS, D_MODEL), jnp.float32),
            pltpu.VMEM((chunks_per_w, CHUNK), jnp.int32),
            [pltpu.VMEM((CHUNK, D_MODEL), jnp.float32) for _ in range(NBUF)],
            [pltpu.SemaphoreType.DMA for _ in range(NBUF)],
            [pltpu.SemaphoreType.DMA for _ in range(NBUF)],
        ],
    )
    def gather_kernel(pe_hbm, idx_hbm, out_hbm, tbl_sp, idx_v, rows, gsem, ssem):
        wid = lax.axis_index("s") * nc + lax.axis_index("c")
        base = wid * rows_per_w

        @pl.when(lax.axis_index("s") == 0)
        def _stage():
            pltpu.sync_copy(pe_hbm, tbl_sp)

        pltpu.sync_copy(idx_hbm.at[wid], idx_v)
        plsc.subcore_barrier()

        def gather(c, b):
            pltpu.async_copy(tbl_sp.at[idx_v.at[c]], rows[b], gsem[b])

        def wait_gather(b):
            pltpu.make_async_copy(tbl_sp.at[idx_v.at[0]], rows[b], gsem[b]).wait()

        def scatter(c, b):
            pltpu.async_copy(
                rows[b], out_hbm.at[pl.ds(base + c * CHUNK, CHUNK)], ssem[b]
            )

        def wait_scatter(b):
            pltpu.make_async_copy(
                rows[b], out_hbm.at[pl.ds(base, CHUNK)], ssem[b]
            ).wait()

        for b in range(NBUF):
            gather(b, b)

        def body(i, _):
            c0 = i * NBUF
            for b in range(NBUF):
                wait_gather(b)
                scatter(c0 + b, b)
            for b in range(NBUF):
                wait_scatter(b)
                gather(c0 + NBUF + b, b)
            return _

        lax.fori_loop(0, ngroups - 1, body, 0)

        for b in range(NBUF):
            wait_gather(b)
            scatter((ngroups - 1) * NBUF + b, b)
        for b in range(NBUF):
            wait_scatter(b)

    def run(pe, idx_flat):
        idx3 = idx_flat.reshape(nw, chunks_per_w, CHUNK)
        return gather_kernel(pe, idx3)

    return run


@jax.jit
def kernel(time, pe):
    b, l = time.shape
    idx_flat = time.astype(jnp.int32).reshape(b * l)
    out = _build(b * l)(pe, idx_flat)
    return out.reshape(b, l, D_MODEL)
